# baseline (speedup 1.0000x reference)
import functools

import jax
import jax.numpy as jnp
from jax import lax
from jax.experimental import pallas as pl
from jax.experimental.pallas import tpu as pltpu
from jax.experimental.pallas import tpu_sc as plsc

_NC = 2
_NS = 16
_NW = _NC * _NS
_CHUNK = 128


def _make_gather(vocab: int, dim: int, batch: int):
    assert batch % _NW == 0
    b_per_w = batch // _NW
    assert b_per_w % _CHUNK == 0
    n_chunks = b_per_w // _CHUNK

    mesh = plsc.VectorSubcoreMesh(core_axis_name="c", subcore_axis_name="s")

    @functools.partial(
        pl.kernel,
        mesh=mesh,
        out_type=jax.ShapeDtypeStruct((batch, dim), jnp.float32),
        scratch_types=[
            pltpu.VMEM((b_per_w,), jnp.int32),
            pltpu.VMEM((b_per_w, dim), jnp.float32),
            pltpu.SemaphoreType.DMA,
        ],
        compiler_params=pltpu.CompilerParams(use_tc_tiling_on_sc=False),
    )
    def gather(table_hbm, ids_hbm, out_hbm, idx_v, rows_v, sem):
        wid = lax.axis_index("s") * _NC + lax.axis_index("c")
        base = wid * b_per_w
        pltpu.sync_copy(ids_hbm.at[pl.ds(base, b_per_w)], idx_v)
        copies = []
        for j in range(n_chunks):
            copies.append(
                pltpu.async_copy(
                    table_hbm.at[idx_v.at[pl.ds(j * _CHUNK, _CHUNK)]],
                    rows_v.at[pl.ds(j * _CHUNK, _CHUNK)],
                    sem,
                )
            )
        for c in copies:
            c.wait()
        pltpu.sync_copy(rows_v, out_hbm.at[pl.ds(base, b_per_w)])

    return gather


@jax.jit
def kernel(input_ids, position_ids, attention_mask, table):
    batch, seq = input_ids.shape
    vocab, dim = table.shape
    flat_ids = input_ids.reshape(batch * seq)
    gather = _make_gather(vocab, dim, batch * seq)
    rows = gather(table, flat_ids)
    inputs_embeds = rows.reshape(batch, seq, dim)
    return (inputs_embeds, position_ids, attention_mask)


# speedup vs baseline: 1.7082x; 1.7082x over previous
import functools

import jax
import jax.numpy as jnp
from jax import lax
from jax.experimental import pallas as pl
from jax.experimental.pallas import tpu as pltpu
from jax.experimental.pallas import tpu_sc as plsc

_NC = 2
_NS = 16
_NW = _NC * _NS
_CHUNK = 256


def _make_gather(vocab: int, dim: int, batch: int):
    assert batch % (_NW * _CHUNK) == 0
    b_per_w = batch // _NW
    n_chunks = b_per_w // _CHUNK

    mesh = plsc.VectorSubcoreMesh(core_axis_name="c", subcore_axis_name="s")

    @functools.partial(
        pl.kernel,
        mesh=mesh,
        out_type=jax.ShapeDtypeStruct((batch, dim), jnp.float32),
        scratch_types=[
            pltpu.VMEM((b_per_w,), jnp.int32),
            pltpu.VMEM((_CHUNK, dim), jnp.float32),
            pltpu.VMEM((_CHUNK, dim), jnp.float32),
            pltpu.SemaphoreType.DMA,
            pltpu.SemaphoreType.DMA,
            pltpu.SemaphoreType.DMA,
            pltpu.SemaphoreType.DMA,
        ],
    )
    def gather(table_hbm, ids_hbm, out_hbm, idx_v, rows_a, rows_b,
               sem_a, sem_b, osem_a, osem_b):
        wid = lax.axis_index("s") * _NC + lax.axis_index("c")
        base = wid * b_per_w
        pltpu.sync_copy(ids_hbm.at[pl.ds(base, b_per_w)], idx_v)
        bufs = ((rows_a, sem_a, osem_a), (rows_b, sem_b, osem_b))

        def fire(c, buf, sem):
            def body(g, _):
                vec = idx_v[pl.ds(c * _CHUNK + g * 16, 16)]
                for l in range(16):
                    pltpu.async_copy(
                        table_hbm.at[pl.ds(vec[l], 1)],
                        buf.at[pl.ds(g * 16 + l, 1)],
                        sem,
                    )
                return ()

            lax.fori_loop(0, _CHUNK // 16, body, ())

        def drain_rows(buf, sem):
            def body(i, _):
                pltpu.make_async_copy(
                    table_hbm.at[pl.ds(0, 1)], buf.at[pl.ds(i, 1)], sem
                ).wait()
                return ()

            lax.fori_loop(0, _CHUNK, body, ())

        def wait_out(c, buf, osem):
            pltpu.make_async_copy(
                buf, out_hbm.at[pl.ds(base + c * _CHUNK, _CHUNK)], osem
            ).wait()

        fire(0, rows_a, sem_a)
        for c in range(n_chunks):
            buf, sem, osem = bufs[c % 2]
            nbuf, nsem, nosem = bufs[(c + 1) % 2]
            drain_rows(buf, sem)
            if c + 1 < n_chunks:
                if c >= 1:
                    wait_out(c - 1, nbuf, nosem)
                fire(c + 1, nbuf, nsem)
            pltpu.async_copy(
                buf, out_hbm.at[pl.ds(base + c * _CHUNK, _CHUNK)], osem
            )
        if n_chunks >= 2:
            c = n_chunks - 2
            buf, _, osem = bufs[c % 2]
            wait_out(c, buf, osem)
        c = n_chunks - 1
        buf, _, osem = bufs[c % 2]
        wait_out(c, buf, osem)

    return gather


@jax.jit
def kernel(input_ids, position_ids, attention_mask, table):
    batch, seq = input_ids.shape
    vocab, dim = table.shape
    flat_ids = input_ids.reshape(batch * seq)
    gather = _make_gather(vocab, dim, batch * seq)
    rows = gather(table, flat_ids)
    inputs_embeds = rows.reshape(batch, seq, dim)
    return (inputs_embeds, position_ids, attention_mask)
